# baseline (speedup 1.0000x reference)
import functools

import jax
import jax.numpy as jnp
from jax import lax
from jax.experimental import pallas as pl
from jax.experimental.pallas import tpu as pltpu
from jax.experimental.pallas import tpu_sc as plsc


def _sc_gather(y, idx_flat, n_pad):
    V, D = y.shape
    NE = idx_flat.shape[0]
    info = plsc.get_sparse_core_info()
    NC, NS = info.num_cores, info.num_subcores
    NW = NC * NS
    assert NE % NW == 0
    per_w = NE // NW
    assert per_w <= n_pad and n_pad % 8 == 0
    C = 40
    NBUF = 5
    K = 2
    assert per_w % (C * NBUF) == 0
    n_chunks = per_w // C

    mesh = plsc.VectorSubcoreMesh(core_axis_name="c", subcore_axis_name="s")

    @functools.partial(
        pl.kernel,
        out_type=jax.ShapeDtypeStruct((NW * n_pad, D), jnp.float32),
        mesh=mesh,
        scratch_types=[
            pltpu.VMEM_SHARED((V, D), jnp.float32),
            pltpu.VMEM((per_w,), jnp.int32),
            pltpu.VMEM((NBUF, C, D), jnp.float32),
            pltpu.SemaphoreType.DMA((NBUF,)),
            pltpu.SemaphoreType.DMA((NBUF,)),
        ],
    )
    def gather_kernel(y_hbm, idx_hbm, out_hbm, tab_sh, idx_all, rows_v, gsem, osem):
        wid = lax.axis_index("s") * NC + lax.axis_index("c")
        base = pl.multiple_of(wid * per_w, 8)
        base_out = pl.multiple_of(wid * n_pad, 8)

        @pl.when(lax.axis_index("s") == 0)
        def _():
            pltpu.sync_copy(y_hbm, tab_sh)

        pltpu.sync_copy(idx_hbm.at[pl.ds(base, per_w)], idx_all)
        plsc.subcore_barrier()

        def start_gather(c, b):
            off = pl.multiple_of(c * C, 8)
            pltpu.make_async_copy(
                tab_sh.at[idx_all.at[pl.ds(off, C)]], rows_v.at[b], gsem.at[b]
            ).start()

        def start_wb(c, b):
            off = pl.multiple_of(base_out + c * C, 8)
            pltpu.make_async_copy(
                rows_v.at[b], out_hbm.at[pl.ds(off, C)], osem.at[b]
            ).start()

        def wait_gather(b):
            pltpu.make_async_copy(
                tab_sh.at[idx_all.at[pl.ds(0, C)]], rows_v.at[b], gsem.at[b]
            ).wait()

        def wait_wb(b):
            pltpu.make_async_copy(
                rows_v.at[b], out_hbm.at[pl.ds(base_out, C)], osem.at[b]
            ).wait()

        for c in range(K):
            start_gather(c, c)

        def outer(o, carry):
            for b in range(NBUF):
                c = o * NBUF + b
                cn = c + K
                bn = (b + K) % NBUF

                @pl.when(jnp.logical_and(cn >= NBUF, cn < n_chunks))
                def _():
                    wait_wb(bn)
                    start_gather(cn, bn)

                @pl.when(jnp.logical_and(cn >= K, cn < NBUF))
                def _():
                    start_gather(cn, bn)

                wait_gather(b)
                start_wb(c, b)
            return carry

        lax.fori_loop(0, n_chunks // NBUF, outer, 0)

        for b in range(NBUF):
            wait_wb(b)

    return gather_kernel(y, idx_flat)


def _edge_block(gx_ref, e_ref, x_ref, ws_ref, wn_ref, we_ref, b_ref, bn, M, E, D):
    F = x_ref.shape[1]
    s = (
        jnp.dot(x_ref[...], ws_ref[...], preferred_element_type=jnp.float32)
        + b_ref[...]
    )
    gy = jnp.dot(
        gx_ref[...].reshape(M * bn, F).astype(jnp.bfloat16),
        wn_ref[...],
        preferred_element_type=jnp.float32,
    ).reshape(M, bn, D)
    t = lax.dot_general(
        e_ref[...], we_ref[...],
        dimension_numbers=(((1,), (0,)), ((), ())),
        preferred_element_type=jnp.float32,
    )
    return gy + s[None, :, :] + t


def _stats_kernel(gx3, nbrP, x, w_self, w_nbr_bf, w_edge, b2, *, bn, n_true):
    M, E, Np = nbrP.shape
    F = gx3.shape[2]
    D = w_self.shape[1]
    grid = Np // bn

    def body(gx_ref, e_ref, x_ref, ws_ref, wn_ref, we_ref, b_ref, st_ref):
        i = pl.program_id(0)

        @pl.when(i == 0)
        def _():
            st_ref[...] = jnp.zeros_like(st_ref)

        g = _edge_block(gx_ref, e_ref, x_ref, ws_ref, wn_ref, we_ref, b_ref,
                        bn, M, E, D)
        node = i * bn + lax.broadcasted_iota(jnp.int32, (1, bn, 1), 1)
        g = jnp.where(node < n_true, g, 0.0)
        gb = g.astype(jnp.bfloat16)
        st_ref[0:1, :] += jnp.sum(g, axis=(0, 1)).reshape(1, D)
        st_ref[1:2, :] += jnp.sum(gb * gb, axis=(0, 1),
                                  dtype=jnp.float32).reshape(1, D)

    return pl.pallas_call(
        body,
        grid=(grid,),
        in_specs=[
            pl.BlockSpec((M, bn, F), lambda i: (0, i, 0)),
            pl.BlockSpec((M, E, bn), lambda i: (0, 0, i)),
            pl.BlockSpec((bn, F), lambda i: (i, 0)),
            pl.BlockSpec((F, D), lambda i: (0, 0)),
            pl.BlockSpec((F, D), lambda i: (0, 0)),
            pl.BlockSpec((E, D), lambda i: (0, 0)),
            pl.BlockSpec((1, D), lambda i: (0, 0)),
        ],
        out_specs=pl.BlockSpec((8, D), lambda i: (0, 0)),
        out_shape=jax.ShapeDtypeStruct((8, D), jnp.float32),
    )(gx3, nbrP, x, w_self, w_nbr_bf, w_edge, b2)


def _apply_kernel(gx3, nbrP, x, w_self, w_nbr_bf, w_edge, b2, st1, gam1,
                  bet1, *, bn, n_true):
    M, E, Np = nbrP.shape
    F = gx3.shape[2]
    D = w_self.shape[1]
    grid = Np // bn

    ne_true = n_true * M

    def body(gx_ref, e_ref, x_ref, ws_ref, wn_ref, we_ref, b_ref,
             st1_ref, gam_ref, bet_ref, ns_ref, st_ref):
        i = pl.program_id(0)

        @pl.when(i == 0)
        def _():
            st_ref[...] = jnp.zeros_like(st_ref)

        mean1 = st1_ref[0:1, :] * (1.0 / ne_true)
        var1 = st1_ref[1:2, :] * (1.0 / ne_true) - mean1 * mean1
        a1 = gam_ref[...] * jax.lax.rsqrt(var1 + 1e-5)
        c1 = bet_ref[...] - mean1 * a1

        g = _edge_block(gx_ref, e_ref, x_ref, ws_ref, wn_ref, we_ref, b_ref,
                        bn, M, E, D)
        node = i * bn + lax.broadcasted_iota(jnp.int32, (1, bn, 1), 1)
        g = jnp.where(node < n_true, g, 0.0)
        g = (g * a1[None, :, :] + c1[None, :, :]).astype(jnp.bfloat16)
        filt = g[:, :, :F]
        core = g[:, :, F:]
        sig = 0.5 * (1.0 + jnp.tanh(0.5 * filt))
        sp = jnp.maximum(core, 0.0) + jnp.log(1.0 + jnp.exp(-jnp.abs(core)))
        ns = jnp.sum(sig * sp, axis=0, dtype=jnp.float32)
        node2 = i * bn + lax.broadcasted_iota(jnp.int32, (bn, 1), 0)
        ns = jnp.where(node2 < n_true, ns, 0.0)
        ns_ref[...] = ns
        st_ref[0:1, :] += jnp.sum(ns, axis=0).reshape(1, F)
        st_ref[1:2, :] += jnp.sum(ns * ns, axis=0).reshape(1, F)

    return pl.pallas_call(
        body,
        grid=(grid,),
        in_specs=[
            pl.BlockSpec((M, bn, F), lambda i: (0, i, 0)),
            pl.BlockSpec((M, E, bn), lambda i: (0, 0, i)),
            pl.BlockSpec((bn, F), lambda i: (i, 0)),
            pl.BlockSpec((F, D), lambda i: (0, 0)),
            pl.BlockSpec((F, D), lambda i: (0, 0)),
            pl.BlockSpec((E, D), lambda i: (0, 0)),
            pl.BlockSpec((1, D), lambda i: (0, 0)),
            pl.BlockSpec((8, D), lambda i: (0, 0)),
            pl.BlockSpec((1, D), lambda i: (0, 0)),
            pl.BlockSpec((1, D), lambda i: (0, 0)),
        ],
        out_specs=(
            pl.BlockSpec((bn, F), lambda i: (i, 0)),
            pl.BlockSpec((8, F), lambda i: (0, 0)),
        ),
        out_shape=(
            jax.ShapeDtypeStruct((Np, F), jnp.float32),
            jax.ShapeDtypeStruct((8, F), jnp.float32),
        ),
    )(gx3, nbrP, x, w_self, w_nbr_bf, w_edge, b2, st1, gam1, bet1)


def _finish_kernel(x, ns, st2, gam2, bet2, n_true):
    Np, F = x.shape

    def body(x_ref, ns_ref, st_ref, gam_ref, bet_ref, o_ref):
        mean2 = st_ref[0:1, :] * (1.0 / n_true)
        var2 = st_ref[1:2, :] * (1.0 / n_true) - mean2 * mean2
        a2 = gam_ref[...] * jax.lax.rsqrt(var2 + 1e-5)
        c2 = bet_ref[...] - mean2 * a2
        v = x_ref[...] + ns_ref[...] * a2 + c2
        o_ref[...] = jnp.maximum(v, 0.0) + jnp.log(1.0 + jnp.exp(-jnp.abs(v)))

    return pl.pallas_call(
        body,
        out_shape=jax.ShapeDtypeStruct((Np, F), jnp.float32),
    )(x, ns, st2, gam2, bet2)


def kernel(atom_in_fea, nbr_fea, nbr_fea_idx, W, b, gamma1, beta1, gamma2, beta2):
    B, A, F = atom_in_fea.shape
    M = nbr_fea.shape[2]
    E = nbr_fea.shape[3]
    N = B * A
    D = 2 * F
    NE = N * M
    eps = 1e-5

    x = atom_in_fea.reshape(N, F)
    bn = 512
    Np = (N + bn - 1) // bn * bn
    nbrP = jnp.transpose(nbr_fea.reshape(N, M, E), (1, 2, 0))
    nbrP = jnp.pad(nbrP, ((0, 0), (0, 0), (0, Np - N)))
    xp = jnp.pad(x, ((0, Np - N), (0, 0)))
    idx_flat = jnp.transpose(nbr_fea_idx.reshape(N, M), (1, 0)).reshape(NE)

    w_self = W[:F]
    w_nbr_bf = W[F : 2 * F].astype(jnp.bfloat16)
    w_edge = W[2 * F :]
    b2 = b.reshape(1, D)

    gx3 = _sc_gather(x, idx_flat, Np).reshape(M, Np, F)

    st1 = _stats_kernel(gx3, nbrP, xp, w_self, w_nbr_bf, w_edge, b2, bn=bn,
                        n_true=N)
    ns, st2 = _apply_kernel(gx3, nbrP, xp, w_self, w_nbr_bf, w_edge, b2, st1,
                            gamma1.reshape(1, D), beta1.reshape(1, D),
                            bn=bn, n_true=N)
    out = _finish_kernel(xp, ns, st2, gamma2.reshape(1, F),
                         beta2.reshape(1, F), N)
    return out[:N].reshape(B, A, F)


# speedup vs baseline: 1.0026x; 1.0026x over previous
import functools

import jax
import jax.numpy as jnp
from jax import lax
from jax.experimental import pallas as pl
from jax.experimental.pallas import tpu as pltpu
from jax.experimental.pallas import tpu_sc as plsc


def _sc_gather(y, idx_flat, n_pad):
    V, D = y.shape
    NE = idx_flat.shape[0]
    info = plsc.get_sparse_core_info()
    NC, NS = info.num_cores, info.num_subcores
    NW = NC * NS
    assert NE % NW == 0
    per_w = NE // NW
    assert per_w <= n_pad and n_pad % 8 == 0
    C = 40
    NBUF = 5
    K = 2
    assert per_w % (C * NBUF) == 0
    n_chunks = per_w // C

    mesh = plsc.VectorSubcoreMesh(core_axis_name="c", subcore_axis_name="s")

    @functools.partial(
        pl.kernel,
        out_type=jax.ShapeDtypeStruct((NW * n_pad, D), jnp.float32),
        mesh=mesh,
        scratch_types=[
            pltpu.VMEM_SHARED((V, D), jnp.float32),
            pltpu.VMEM((per_w,), jnp.int32),
            pltpu.VMEM((NBUF, C, D), jnp.float32),
            pltpu.SemaphoreType.DMA((NBUF,)),
            pltpu.SemaphoreType.DMA((NBUF,)),
        ],
    )
    def gather_kernel(y_hbm, idx_hbm, out_hbm, tab_sh, idx_all, rows_v, gsem, osem):
        wid = lax.axis_index("s") * NC + lax.axis_index("c")
        base = pl.multiple_of(wid * per_w, 8)
        base_out = pl.multiple_of(wid * n_pad, 8)

        @pl.when(lax.axis_index("s") == 0)
        def _():
            pltpu.sync_copy(y_hbm, tab_sh)

        pltpu.sync_copy(idx_hbm.at[pl.ds(base, per_w)], idx_all)
        plsc.subcore_barrier()

        def start_gather(c, b):
            off = pl.multiple_of(c * C, 8)
            pltpu.make_async_copy(
                tab_sh.at[idx_all.at[pl.ds(off, C)]], rows_v.at[b], gsem.at[b]
            ).start()

        def start_wb(c, b):
            off = pl.multiple_of(base_out + c * C, 8)
            pltpu.make_async_copy(
                rows_v.at[b], out_hbm.at[pl.ds(off, C)], osem.at[b]
            ).start()

        def wait_gather(b):
            pltpu.make_async_copy(
                tab_sh.at[idx_all.at[pl.ds(0, C)]], rows_v.at[b], gsem.at[b]
            ).wait()

        def wait_wb(b):
            pltpu.make_async_copy(
                rows_v.at[b], out_hbm.at[pl.ds(base_out, C)], osem.at[b]
            ).wait()

        for c in range(K):
            start_gather(c, c)

        def outer(o, carry):
            for b in range(NBUF):
                c = o * NBUF + b
                cn = c + K
                bn = (b + K) % NBUF

                @pl.when(jnp.logical_and(cn >= NBUF, cn < n_chunks))
                def _():
                    wait_wb(bn)
                    start_gather(cn, bn)

                @pl.when(jnp.logical_and(cn >= K, cn < NBUF))
                def _():
                    start_gather(cn, bn)

                wait_gather(b)
                start_wb(c, b)
            return carry

        lax.fori_loop(0, n_chunks // NBUF, outer, 0)

        for b in range(NBUF):
            wait_wb(b)

    return gather_kernel(y, idx_flat)


def _edge_block(gx_ref, e_ref, x_ref, ws_ref, wn_ref, we_ref, b_ref, bn, M, E, D):
    F = x_ref.shape[1]
    s = (
        jnp.dot(x_ref[...], ws_ref[...], preferred_element_type=jnp.float32)
        + b_ref[...]
    )
    gy = jnp.dot(
        gx_ref[...].reshape(M * bn, F).astype(jnp.bfloat16),
        wn_ref[...],
        preferred_element_type=jnp.float32,
    ).reshape(M, bn, D)
    t = lax.dot_general(
        e_ref[...], we_ref[...],
        dimension_numbers=(((1,), (0,)), ((), ())),
        preferred_element_type=jnp.float32,
    )
    return gy + s[None, :, :] + t


def _stats_kernel(gx3, nbrP, x, w_self, w_nbr_bf, w_edge, b2, *, bn, n_true):
    M, E, Np = nbrP.shape
    F = gx3.shape[2]
    D = w_self.shape[1]
    grid = Np // bn

    def body(gx_ref, e_ref, x_ref, ws_ref, wn_ref, we_ref, b_ref, st_ref):
        i = pl.program_id(0)

        @pl.when(i == 0)
        def _():
            st_ref[...] = jnp.zeros_like(st_ref)

        g = _edge_block(gx_ref, e_ref, x_ref, ws_ref, wn_ref, we_ref, b_ref,
                        bn, M, E, D)
        node = i * bn + lax.broadcasted_iota(jnp.int32, (1, bn, 1), 1)
        g = jnp.where(node < n_true, g, 0.0)
        st_ref[0:1, :] += jnp.sum(g, axis=(0, 1)).reshape(1, D)
        st_ref[1:2, :] += jnp.sum(g * g, axis=(0, 1)).reshape(1, D)

    return pl.pallas_call(
        body,
        grid=(grid,),
        in_specs=[
            pl.BlockSpec((M, bn, F), lambda i: (0, i, 0)),
            pl.BlockSpec((M, E, bn), lambda i: (0, 0, i)),
            pl.BlockSpec((bn, F), lambda i: (i, 0)),
            pl.BlockSpec((F, D), lambda i: (0, 0)),
            pl.BlockSpec((F, D), lambda i: (0, 0)),
            pl.BlockSpec((E, D), lambda i: (0, 0)),
            pl.BlockSpec((1, D), lambda i: (0, 0)),
        ],
        out_specs=pl.BlockSpec((8, D), lambda i: (0, 0)),
        out_shape=jax.ShapeDtypeStruct((8, D), jnp.float32),
    )(gx3, nbrP, x, w_self, w_nbr_bf, w_edge, b2)


def _apply_kernel(gx3, nbrP, x, w_self, w_nbr_bf, w_edge, b2, st1, gam1,
                  bet1, *, bn, n_true):
    M, E, Np = nbrP.shape
    F = gx3.shape[2]
    D = w_self.shape[1]
    grid = Np // bn

    ne_true = n_true * M

    def body(gx_ref, e_ref, x_ref, ws_ref, wn_ref, we_ref, b_ref,
             st1_ref, gam_ref, bet_ref, ns_ref, st_ref):
        i = pl.program_id(0)

        @pl.when(i == 0)
        def _():
            st_ref[...] = jnp.zeros_like(st_ref)

        mean1 = st1_ref[0:1, :] * (1.0 / ne_true)
        var1 = st1_ref[1:2, :] * (1.0 / ne_true) - mean1 * mean1
        a1 = gam_ref[...] * jax.lax.rsqrt(var1 + 1e-5)
        c1 = bet_ref[...] - mean1 * a1

        g = _edge_block(gx_ref, e_ref, x_ref, ws_ref, wn_ref, we_ref, b_ref,
                        bn, M, E, D)
        node = i * bn + lax.broadcasted_iota(jnp.int32, (1, bn, 1), 1)
        g = jnp.where(node < n_true, g, 0.0)
        g = (g * a1[None, :, :] + c1[None, :, :]).astype(jnp.bfloat16)
        filt = g[:, :, :F]
        core = g[:, :, F:]
        sig = 0.5 * (1.0 + jnp.tanh(0.5 * filt))
        sp = jnp.maximum(core, 0.0) + jnp.log(1.0 + jnp.exp(-jnp.abs(core)))
        ns = jnp.sum(sig * sp, axis=0, dtype=jnp.float32)
        node2 = i * bn + lax.broadcasted_iota(jnp.int32, (bn, 1), 0)
        ns = jnp.where(node2 < n_true, ns, 0.0)
        ns_ref[...] = ns
        st_ref[0:1, :] += jnp.sum(ns, axis=0).reshape(1, F)
        st_ref[1:2, :] += jnp.sum(ns * ns, axis=0).reshape(1, F)

    return pl.pallas_call(
        body,
        grid=(grid,),
        in_specs=[
            pl.BlockSpec((M, bn, F), lambda i: (0, i, 0)),
            pl.BlockSpec((M, E, bn), lambda i: (0, 0, i)),
            pl.BlockSpec((bn, F), lambda i: (i, 0)),
            pl.BlockSpec((F, D), lambda i: (0, 0)),
            pl.BlockSpec((F, D), lambda i: (0, 0)),
            pl.BlockSpec((E, D), lambda i: (0, 0)),
            pl.BlockSpec((1, D), lambda i: (0, 0)),
            pl.BlockSpec((8, D), lambda i: (0, 0)),
            pl.BlockSpec((1, D), lambda i: (0, 0)),
            pl.BlockSpec((1, D), lambda i: (0, 0)),
        ],
        out_specs=(
            pl.BlockSpec((bn, F), lambda i: (i, 0)),
            pl.BlockSpec((8, F), lambda i: (0, 0)),
        ),
        out_shape=(
            jax.ShapeDtypeStruct((Np, F), jnp.float32),
            jax.ShapeDtypeStruct((8, F), jnp.float32),
        ),
    )(gx3, nbrP, x, w_self, w_nbr_bf, w_edge, b2, st1, gam1, bet1)


def _finish_kernel(x, ns, st2, gam2, bet2, n_true):
    Np, F = x.shape

    def body(x_ref, ns_ref, st_ref, gam_ref, bet_ref, o_ref):
        mean2 = st_ref[0:1, :] * (1.0 / n_true)
        var2 = st_ref[1:2, :] * (1.0 / n_true) - mean2 * mean2
        a2 = gam_ref[...] * jax.lax.rsqrt(var2 + 1e-5)
        c2 = bet_ref[...] - mean2 * a2
        v = x_ref[...] + ns_ref[...] * a2 + c2
        o_ref[...] = jnp.maximum(v, 0.0) + jnp.log(1.0 + jnp.exp(-jnp.abs(v)))

    return pl.pallas_call(
        body,
        out_shape=jax.ShapeDtypeStruct((Np, F), jnp.float32),
    )(x, ns, st2, gam2, bet2)


def kernel(atom_in_fea, nbr_fea, nbr_fea_idx, W, b, gamma1, beta1, gamma2, beta2):
    B, A, F = atom_in_fea.shape
    M = nbr_fea.shape[2]
    E = nbr_fea.shape[3]
    N = B * A
    D = 2 * F
    NE = N * M

    x = atom_in_fea.reshape(N, F)
    bn = 512
    Np = (N + bn - 1) // bn * bn
    nbrP = jnp.transpose(nbr_fea.reshape(N, M, E), (1, 2, 0))
    nbrP = jnp.pad(nbrP, ((0, 0), (0, 0), (0, Np - N)))
    xp = jnp.pad(x, ((0, Np - N), (0, 0)))
    idx_flat = jnp.transpose(nbr_fea_idx.reshape(N, M), (1, 0)).reshape(NE)

    w_self = W[:F]
    w_nbr_bf = W[F : 2 * F].astype(jnp.bfloat16)
    w_edge = W[2 * F :]
    b2 = b.reshape(1, D)

    gx3 = _sc_gather(x, idx_flat, Np).reshape(M, Np, F)

    st1 = _stats_kernel(gx3, nbrP, xp, w_self, w_nbr_bf, w_edge, b2, bn=bn,
                        n_true=N)
    ns, st2 = _apply_kernel(gx3, nbrP, xp, w_self, w_nbr_bf, w_edge, b2, st1,
                            gamma1.reshape(1, D), beta1.reshape(1, D),
                            bn=bn, n_true=N)
    out = _finish_kernel(xp, ns, st2, gamma2.reshape(1, F),
                         beta2.reshape(1, F), N)
    return out[:N].reshape(B, A, F)
